# baseline (speedup 1.0000x reference)
import jax
import jax.numpy as jnp
from jax.experimental import pallas as pl
from jax.experimental.pallas import tpu as pltpu

_EPS = 1e-7
_ROWS_PER_BLOCK = 8


def _nll_body(p_ref, o_ref, out_ref):
    i = pl.program_id(0)

    @pl.when(i == 0)
    def _init():
        out_ref[0, 0] = 0.0

    p = p_ref[...]
    o = o_ref[...]
    t = jnp.where(o, -jnp.log(p + _EPS), 0.0)
    out_ref[0, 0] += jnp.sum(t)


def kernel(predicted, observed):
    B, N = predicted.shape
    grid = (B // _ROWS_PER_BLOCK,)
    out = pl.pallas_call(
        _nll_body,
        grid=grid,
        in_specs=[
            pl.BlockSpec((_ROWS_PER_BLOCK, N), lambda i: (i, 0)),
            pl.BlockSpec((_ROWS_PER_BLOCK, N), lambda i: (i, 0)),
        ],
        out_specs=pl.BlockSpec(memory_space=pltpu.SMEM),
        out_shape=jax.ShapeDtypeStruct((1, 1), jnp.float32),
    )(predicted, observed)
    return out[0, 0] / B
